# Initial kernel scaffold; baseline (speedup 1.0000x reference)
import jax
import jax.numpy as jnp
from jax.experimental import pallas as pl


def kernel(pred_ccoords, pred_beta, pred_dist):
    raise NotImplementedError("write your pallas kernel here")


# speedup vs baseline: 40.5744x; 40.5744x over previous
import jax
import jax.numpy as jnp
from jax import lax
from jax.experimental import pallas as pl
from jax.experimental.pallas import tpu as pltpu

_BETA_THRESHOLD = 0.3
_DISTANCE_THRESHOLD = 0.5
_N = 20000
_ROWS = 8
_COLS = 2560
_P = _ROWS * _COLS


def _greedy_body(cx_ref, cy_ref, cz_ref, t2_ref, b_ref,
                 assign_ref, alpha_ref, ox_ref, oy_ref, oz_ref, mb_ref):
    shape = (_ROWS, _COLS)
    neg1 = jnp.full(shape, -1, jnp.int32)
    assign_ref[...] = neg1
    alpha_ref[...] = neg1
    zero = jnp.zeros(shape, jnp.float32)
    ox_ref[...] = zero
    oy_ref[...] = zero
    oz_ref[...] = zero
    mb_ref[...] = b_ref[...]

    row = lax.broadcasted_iota(jnp.int32, shape, 0)
    col = lax.broadcasted_iota(jnp.int32, shape, 1)
    idx = row * _COLS + col

    cx = cx_ref[...]
    cy = cy_ref[...]
    cz = cz_ref[...]
    t2 = t2_ref[...]

    def cond_fn(carry):
        _, m = carry
        return m > _BETA_THRESHOLD

    def body_fn(carry):
        k, m = carry
        mb = mb_ref[...]
        a = jnp.min(jnp.where(mb >= m, idx, _P))
        sel = idx == a
        ninf = jnp.float32(-jnp.inf)
        cxa = jnp.max(jnp.where(sel, cx, ninf))
        cya = jnp.max(jnp.where(sel, cy, ninf))
        cza = jnp.max(jnp.where(sel, cz, ninf))
        t2a = jnp.max(jnp.where(sel, t2, ninf))
        dx = cx - cxa
        dy = cy - cya
        dz = cz - cza
        d2 = (dx * dx + dy * dy) + dz * dz
        within = (d2 <= t2a) & (assign_ref[...] < 0)
        assign_ref[...] = jnp.where(within, k, assign_ref[...])
        alpha_ref[...] = jnp.where(within, a, alpha_ref[...])
        ox_ref[...] = jnp.where(within, cxa, ox_ref[...])
        oy_ref[...] = jnp.where(within, cya, oy_ref[...])
        oz_ref[...] = jnp.where(within, cza, oz_ref[...])
        mb2 = jnp.where(within, jnp.float32(-2.0), mb)
        mb_ref[...] = mb2
        return k + jnp.int32(1), jnp.max(mb2)

    m0 = jnp.max(b_ref[...])
    lax.while_loop(cond_fn, body_fn, (jnp.int32(0), m0))


def _thr2(radius):
    r2 = radius * radius
    bits = r2.view(jnp.int32)
    best = jnp.full_like(r2, -1.0)
    for j in range(-8, 9):
        c = jnp.maximum(bits + j, 0).view(jnp.float32)
        keep = jnp.sqrt(c) <= radius
        best = jnp.maximum(best, jnp.where(keep, c, -1.0))
    return best


@jax.jit
def kernel(pred_ccoords, pred_beta, pred_dist):
    pad = _P - _N
    coords = jnp.pad(pred_ccoords, ((0, pad), (0, 0)), constant_values=1e9)
    cx = coords[:, 0].reshape(_ROWS, _COLS)
    cy = coords[:, 1].reshape(_ROWS, _COLS)
    cz = coords[:, 2].reshape(_ROWS, _COLS)
    beta = jnp.pad(pred_beta.reshape(-1), (0, pad), constant_values=-1.0)
    beta = beta.reshape(_ROWS, _COLS)
    radius = pred_dist.reshape(-1) * _DISTANCE_THRESHOLD
    t2 = jnp.pad(_thr2(radius), (0, pad), constant_values=-1.0)
    t2 = t2.reshape(_ROWS, _COLS)

    shape = (_ROWS, _COLS)
    out_shapes = (
        jax.ShapeDtypeStruct(shape, jnp.int32),
        jax.ShapeDtypeStruct(shape, jnp.int32),
        jax.ShapeDtypeStruct(shape, jnp.float32),
        jax.ShapeDtypeStruct(shape, jnp.float32),
        jax.ShapeDtypeStruct(shape, jnp.float32),
    )
    assign, alpha, ox, oy, oz = pl.pallas_call(
        _greedy_body,
        out_shape=out_shapes,
        scratch_shapes=[pltpu.VMEM(shape, jnp.float32)],
    )(cx, cy, cz, t2, beta)

    assign = assign.reshape(-1)[:_N]
    alpha = alpha.reshape(-1)[:_N]
    cond = jnp.stack([ox.reshape(-1)[:_N], oy.reshape(-1)[:_N],
                      oz.reshape(-1)[:_N]], axis=-1)
    return assign, alpha, cond
